# initial kernel scaffold (unmeasured)
import jax
import jax.numpy as jnp
from jax import lax
from jax.experimental import pallas as pl
from jax.experimental.pallas import tpu as pltpu

N_DEV = 8
B, SQ, SKV, D = 2, 256, 512, 768
HQ_PER = 8
DH = 64
DM = HQ_PER * DH
SCALE = 0.125


def kernel(x, Wq, Wo, K_ext, V_ext):
    def body(x_ref, wq_ref, wo_ref, k_hbm, v_hbm, out_ref,
             k_vmem, v_vmem, attn_ref, acc_ref, comm_ref,
             send_sems, recv_sems, kv_sem):
        my = lax.axis_index("i")
        left = (my - 1) % N_DEV
        right = (my + 1) % N_DEV

        h0 = my * HQ_PER
        k_copy = pltpu.make_async_copy(
            k_hbm.at[:, :, pl.ds(h0, HQ_PER), :], k_vmem, kv_sem)
        v_copy = pltpu.make_async_copy(
            v_hbm.at[:, :, pl.ds(h0, HQ_PER), :], v_vmem, kv_sem)
        k_copy.start()
        v_copy.start()

        barrier = pltpu.get_barrier_semaphore()
        for nbr in (left, right):
            pl.semaphore_signal(
                barrier, inc=1,
                device_id=(nbr,), device_id_type=pl.DeviceIdType.MESH)
        pl.semaphore_wait(barrier, 2)

        k_copy.wait()
        v_copy.wait()

        for b in range(B):
            qb = lax.dot_general(
                x_ref[b], wq_ref[...], (((1,), (0,)), ((), ())),
                preferred_element_type=jnp.float32)
            for h in range(HQ_PER):
                q = qb[:, h * DH:(h + 1) * DH]
                k = k_vmem[b, :, h, :]
                v = v_vmem[b, :, h, :]
                s = lax.dot_general(
                    q, k, (((1,), (1,)), ((), ())),
                    preferred_element_type=jnp.float32) * SCALE
                m = jnp.max(s, axis=1, keepdims=True)
                p = jnp.exp(s - m)
                l = jnp.sum(p, axis=1, keepdims=True)
                o = lax.dot_general(
                    p, v, (((1,), (0,)), ((), ())),
                    preferred_element_type=jnp.float32) / l
                attn_ref[b * SQ:(b + 1) * SQ, h * DH:(h + 1) * DH] = o

        partial = lax.dot_general(
            attn_ref[...], wo_ref[...], (((1,), (0,)), ((), ())),
            preferred_element_type=jnp.float32)
        acc_ref[...] = partial
        comm_ref[0] = partial

        for g in range(N_DEV - 1):
            rdma = pltpu.make_async_remote_copy(
                src_ref=comm_ref.at[g],
                dst_ref=comm_ref.at[g + 1],
                send_sem=send_sems.at[g],
                recv_sem=recv_sems.at[g],
                device_id=(right,),
                device_id_type=pl.DeviceIdType.MESH)
            rdma.start()
            rdma.wait()
            acc_ref[...] += comm_ref[g + 1]

        for b in range(B):
            out_ref[b] = acc_ref[b * SQ:(b + 1) * SQ, :]

    return pl.pallas_call(
        body,
        out_shape=jax.ShapeDtypeStruct((B, SQ, D), jnp.float32),
        in_specs=[
            pl.BlockSpec(memory_space=pltpu.VMEM),
            pl.BlockSpec(memory_space=pltpu.VMEM),
            pl.BlockSpec(memory_space=pltpu.VMEM),
            pl.BlockSpec(memory_space=pltpu.ANY),
            pl.BlockSpec(memory_space=pltpu.ANY),
        ],
        out_specs=pl.BlockSpec(memory_space=pltpu.VMEM),
        scratch_shapes=[
            pltpu.VMEM((B, SKV, HQ_PER, DH), jnp.float32),
            pltpu.VMEM((B, SKV, HQ_PER, DH), jnp.float32),
            pltpu.VMEM((B * SQ, DM), jnp.float32),
            pltpu.VMEM((B * SQ, D), jnp.float32),
            pltpu.VMEM((N_DEV, B * SQ, D), jnp.float32),
            pltpu.SemaphoreType.DMA((N_DEV - 1,)),
            pltpu.SemaphoreType.DMA((N_DEV - 1,)),
            pltpu.SemaphoreType.DMA,
        ],
        compiler_params=pltpu.CompilerParams(collective_id=0),
    )(x, Wq, Wo, K_ext, V_ext)


# baseline (device time: 209168 ns/iter reference)
import jax
import jax.numpy as jnp
from jax import lax
from jax.experimental import pallas as pl
from jax.experimental.pallas import tpu as pltpu

N_DEV = 8
B, SQ, SKV, D = 2, 256, 512, 768
HQ_PER = 8
DH = 64
DM = HQ_PER * DH
SCALE = 0.125


def kernel(x, Wq, Wo, K_ext, V_ext):
    def body(x_ref, wq_ref, wo_ref, k_hbm, v_hbm, out_ref,
             k_vmem, v_vmem, attn_ref, acc_ref, comm_ref,
             send_sems, recv_sems, kv_sem):
        my = lax.axis_index("i")
        left = (my - 1) % N_DEV
        right = (my + 1) % N_DEV

        h0 = my * HQ_PER
        k_copy = pltpu.make_async_copy(
            k_hbm.at[:, :, pl.ds(h0, HQ_PER), :], k_vmem, kv_sem)
        v_copy = pltpu.make_async_copy(
            v_hbm.at[:, :, pl.ds(h0, HQ_PER), :], v_vmem, kv_sem)
        k_copy.start()
        v_copy.start()

        barrier = pltpu.get_barrier_semaphore()
        for nbr in (left, right):
            pl.semaphore_signal(
                barrier, inc=1,
                device_id=(nbr,), device_id_type=pl.DeviceIdType.MESH)
        pl.semaphore_wait(barrier, 2)

        k_copy.wait()
        v_copy.wait()

        for b in range(B):
            qb = lax.dot_general(
                x_ref[b], wq_ref[...], (((1,), (0,)), ((), ())),
                preferred_element_type=jnp.float32)
            for h in range(HQ_PER):
                q = qb[:, h * DH:(h + 1) * DH]
                k = k_vmem[b, :, h, :]
                v = v_vmem[b, :, h, :]
                s = lax.dot_general(
                    q, k, (((1,), (1,)), ((), ())),
                    preferred_element_type=jnp.float32) * SCALE
                m = jnp.max(s, axis=1, keepdims=True)
                p = jnp.exp(s - m)
                l = jnp.sum(p, axis=1, keepdims=True)
                o = lax.dot_general(
                    p, v, (((1,), (0,)), ((), ())),
                    preferred_element_type=jnp.float32) / l
                attn_ref[b * SQ:(b + 1) * SQ, h * DH:(h + 1) * DH] = o

        partial = lax.dot_general(
            attn_ref[...], wo_ref[...], (((1,), (0,)), ((), ())),
            preferred_element_type=jnp.float32)
        acc_ref[...] = partial
        comm_ref[0] = partial

        for g in range(N_DEV - 1):
            rdma = pltpu.make_async_remote_copy(
                src_ref=comm_ref.at[g],
                dst_ref=comm_ref.at[g + 1],
                send_sem=send_sems.at[g],
                recv_sem=recv_sems.at[g],
                device_id=(right,),
                device_id_type=pl.DeviceIdType.MESH)
            rdma.start()
            rdma.wait()
            acc_ref[...] += comm_ref[g + 1]

        for b in range(B):
            out_ref[b] = acc_ref[b * SQ:(b + 1) * SQ, :]

    return pl.pallas_call(
        body,
        out_shape=jax.ShapeDtypeStruct((B, SQ, D), jnp.float32),
        in_specs=[
            pl.BlockSpec(memory_space=pltpu.VMEM),
            pl.BlockSpec(memory_space=pltpu.VMEM),
            pl.BlockSpec(memory_space=pltpu.VMEM),
            pl.BlockSpec(memory_space=pl.ANY),
            pl.BlockSpec(memory_space=pl.ANY),
        ],
        out_specs=pl.BlockSpec(memory_space=pltpu.VMEM),
        scratch_shapes=[
            pltpu.VMEM((B, SKV, HQ_PER, DH), jnp.float32),
            pltpu.VMEM((B, SKV, HQ_PER, DH), jnp.float32),
            pltpu.VMEM((B * SQ, DM), jnp.float32),
            pltpu.VMEM((B * SQ, D), jnp.float32),
            pltpu.VMEM((N_DEV, B * SQ, D), jnp.float32),
            pltpu.SemaphoreType.DMA((N_DEV - 1,)),
            pltpu.SemaphoreType.DMA((N_DEV - 1,)),
            pltpu.SemaphoreType.DMA,
        ],
        compiler_params=pltpu.CompilerParams(collective_id=0),
    )(x, Wq, Wo, K_ext, V_ext)


# device time: 129148 ns/iter; 1.6196x vs baseline; 1.6196x over previous
import jax
import jax.numpy as jnp
from jax import lax
from jax.experimental import pallas as pl
from jax.experimental.pallas import tpu as pltpu

N_DEV = 8
B, SQ, SKV, D = 2, 256, 512, 768
HQ_PER = 8
DH = 64
DM = HQ_PER * DH
SCALE = 0.125
ROWS = B * SQ

_MESH = pl.DeviceIdType.MESH


def kernel(x, Wq, Wo, K_ext, V_ext):
    def body(x_ref, wq_ref, wo_ref, k_hbm, v_hbm, out_ref,
             k_vmem, v_vmem, attn_ref, acc_ref, sbuf, cbuf,
             send_sems, recv_sems, kv_sems):
        my = lax.axis_index("i")
        h0 = my * HQ_PER

        kv_copies = []
        for b in range(B):
            for h in range(HQ_PER):
                ck = pltpu.make_async_copy(
                    k_hbm.at[b, :, h0 + h, :], k_vmem.at[b, h],
                    kv_sems.at[0])
                cv = pltpu.make_async_copy(
                    v_hbm.at[b, :, h0 + h, :], v_vmem.at[b, h],
                    kv_sems.at[1])
                ck.start()
                cv.start()
                kv_copies.append(ck)
                kv_copies.append(cv)

        x2 = x_ref[...].reshape(ROWS, D).astype(jnp.bfloat16)
        q_all = lax.dot_general(
            x2, wq_ref[...].astype(jnp.bfloat16), (((1,), (0,)), ((), ())),
            preferred_element_type=jnp.float32).astype(jnp.bfloat16)

        for c in kv_copies:
            c.wait()

        for b in range(B):
            for h in range(HQ_PER):
                q_bh = q_all[b * SQ:(b + 1) * SQ, h * DH:(h + 1) * DH]
                k_bh = k_vmem[b, h].astype(jnp.bfloat16)
                v_bh = v_vmem[b, h].astype(jnp.bfloat16)
                s = lax.dot_general(
                    q_bh, k_bh, (((1,), (1,)), ((), ())),
                    preferred_element_type=jnp.float32) * SCALE
                m = jnp.max(s, axis=1, keepdims=True)
                p = jnp.exp(s - m)
                l = jnp.sum(p, axis=1, keepdims=True)
                o = lax.dot_general(
                    p.astype(jnp.bfloat16), v_bh, (((1,), (0,)), ((), ())),
                    preferred_element_type=jnp.float32)
                attn_ref[pl.ds(b * SQ, SQ), pl.ds(h * DH, DH)] = (
                    o / l).astype(jnp.bfloat16)

        acc_ref[...] = lax.dot_general(
            attn_ref[...], wo_ref[...].astype(jnp.bfloat16),
            (((1,), (0,)), ((), ())),
            preferred_element_type=jnp.float32)

        sbuf[...] = acc_ref[...].astype(jnp.bfloat16)
        sends = []
        for d in range(1, N_DEV):
            peer = lax.rem(my + d, N_DEV)
            rdma = pltpu.make_async_remote_copy(
                src_ref=sbuf, dst_ref=cbuf.at[d - 1],
                send_sem=send_sems.at[d - 1], recv_sem=recv_sems.at[d - 1],
                device_id=(peer,), device_id_type=_MESH)
            rdma.start()
            sends.append(rdma)

        for e in range(1, N_DEV):
            recv = pltpu.make_async_remote_copy(
                src_ref=sbuf, dst_ref=cbuf.at[e - 1],
                send_sem=send_sems.at[e - 1], recv_sem=recv_sems.at[e - 1],
                device_id=(my,), device_id_type=_MESH)
            recv.wait_recv()
            acc_ref[...] += cbuf[e - 1].astype(jnp.float32)

        for b in range(B):
            out_ref[b] = acc_ref[pl.ds(b * SQ, SQ), :]

        for rdma in sends:
            rdma.wait_send()

    return pl.pallas_call(
        body,
        out_shape=jax.ShapeDtypeStruct((B, SQ, D), jnp.float32),
        in_specs=[
            pl.BlockSpec(memory_space=pltpu.MemorySpace.VMEM),
            pl.BlockSpec(memory_space=pltpu.MemorySpace.VMEM),
            pl.BlockSpec(memory_space=pltpu.MemorySpace.VMEM),
            pl.BlockSpec(memory_space=pl.ANY),
            pl.BlockSpec(memory_space=pl.ANY),
        ],
        out_specs=pl.BlockSpec(memory_space=pltpu.MemorySpace.VMEM),
        scratch_shapes=[
            pltpu.VMEM((B, HQ_PER, SKV, DH), jnp.float32),
            pltpu.VMEM((B, HQ_PER, SKV, DH), jnp.float32),
            pltpu.VMEM((ROWS, DM), jnp.bfloat16),
            pltpu.VMEM((ROWS, D), jnp.float32),
            pltpu.VMEM((ROWS, D), jnp.bfloat16),
            pltpu.VMEM((N_DEV - 1, ROWS, D), jnp.bfloat16),
            pltpu.SemaphoreType.DMA((N_DEV - 1,)),
            pltpu.SemaphoreType.DMA((N_DEV - 1,)),
            pltpu.SemaphoreType.DMA((2,)),
        ],
    )(x, Wq, Wo, K_ext, V_ext)


# device time: 72367 ns/iter; 2.8904x vs baseline; 1.7846x over previous
import jax
import jax.numpy as jnp
from jax import lax
from jax.experimental import pallas as pl
from jax.experimental.pallas import tpu as pltpu

N_DEV = 8
B, SQ, SKV, D = 2, 256, 512, 768
HQ_PER = 8
DH = 64
DM = HQ_PER * DH
SCALE = 0.125
ROWS = B * SQ

_MESH = pl.DeviceIdType.MESH
_ALLREDUCE = True


def kernel(x, Wq, Wo, K_ext, V_ext):
    def body(x_ref, wq_ref, wo_ref, k_hbm, v_hbm, out_ref,
             k_vmem, v_vmem, attn_ref, acc_ref, sbuf, cbuf,
             send_sems, recv_sems, kv_sems):
        my = lax.axis_index("i")
        h0 = my * HQ_PER

        kv_copies = []
        for b in range(B):
            for h in range(HQ_PER):
                ck = pltpu.make_async_copy(
                    k_hbm.at[b, :, h0 + h, :], k_vmem.at[b, h],
                    kv_sems.at[0])
                cv = pltpu.make_async_copy(
                    v_hbm.at[b, :, h0 + h, :], v_vmem.at[b, h],
                    kv_sems.at[1])
                ck.start()
                cv.start()
                kv_copies.append(ck)
                kv_copies.append(cv)

        x2 = x_ref[...].reshape(ROWS, D).astype(jnp.bfloat16)
        q_all = lax.dot_general(
            x2, wq_ref[...].astype(jnp.bfloat16), (((1,), (0,)), ((), ())),
            preferred_element_type=jnp.float32).astype(jnp.bfloat16)

        for c in kv_copies:
            c.wait()

        for b in range(B):
            for h in range(HQ_PER):
                q_bh = q_all[b * SQ:(b + 1) * SQ, h * DH:(h + 1) * DH]
                k_bh = k_vmem[b, h].astype(jnp.bfloat16)
                v_bh = v_vmem[b, h].astype(jnp.bfloat16)
                s = lax.dot_general(
                    q_bh, k_bh, (((1,), (1,)), ((), ())),
                    preferred_element_type=jnp.float32) * SCALE
                m = jnp.max(s, axis=1, keepdims=True)
                p = jnp.exp(s - m)
                l = jnp.sum(p, axis=1, keepdims=True)
                o = lax.dot_general(
                    p.astype(jnp.bfloat16), v_bh, (((1,), (0,)), ((), ())),
                    preferred_element_type=jnp.float32)
                attn_ref[pl.ds(b * SQ, SQ), pl.ds(h * DH, DH)] = (
                    o / l).astype(jnp.bfloat16)

        acc_ref[...] = lax.dot_general(
            attn_ref[...], wo_ref[...].astype(jnp.bfloat16),
            (((1,), (0,)), ((), ())),
            preferred_element_type=jnp.float32)

        if not _ALLREDUCE:
            for b in range(B):
                out_ref[b] = acc_ref[b * SQ:(b + 1) * SQ, :]
            return
        sbuf[...] = acc_ref[...].astype(jnp.bfloat16)
        sends = []
        for d in range(1, N_DEV):
            peer = lax.rem(my + d, N_DEV)
            rdma = pltpu.make_async_remote_copy(
                src_ref=sbuf, dst_ref=cbuf.at[d - 1],
                send_sem=send_sems.at[d - 1], recv_sem=recv_sems.at[d - 1],
                device_id=(peer,), device_id_type=_MESH)
            rdma.start()
            sends.append(rdma)

        for e in range(1, N_DEV):
            recv = pltpu.make_async_remote_copy(
                src_ref=sbuf, dst_ref=cbuf.at[e - 1],
                send_sem=send_sems.at[e - 1], recv_sem=recv_sems.at[e - 1],
                device_id=(my,), device_id_type=_MESH)
            recv.wait_recv()
            acc_ref[...] += cbuf[e - 1].astype(jnp.float32)

        for b in range(B):
            out_ref[b] = acc_ref[pl.ds(b * SQ, SQ), :]

        for rdma in sends:
            rdma.wait_send()

    return pl.pallas_call(
        body,
        out_shape=jax.ShapeDtypeStruct((B, SQ, D), jnp.float32),
        in_specs=[
            pl.BlockSpec(memory_space=pltpu.MemorySpace.VMEM),
            pl.BlockSpec(memory_space=pltpu.MemorySpace.VMEM),
            pl.BlockSpec(memory_space=pltpu.MemorySpace.VMEM),
            pl.BlockSpec(memory_space=pl.ANY),
            pl.BlockSpec(memory_space=pl.ANY),
        ],
        out_specs=pl.BlockSpec(memory_space=pltpu.MemorySpace.VMEM),
        scratch_shapes=[
            pltpu.VMEM((B, HQ_PER, SKV, DH), jnp.float32),
            pltpu.VMEM((B, HQ_PER, SKV, DH), jnp.float32),
            pltpu.VMEM((ROWS, DM), jnp.bfloat16),
            pltpu.VMEM((ROWS, D), jnp.float32),
            pltpu.VMEM((ROWS, D), jnp.bfloat16),
            pltpu.VMEM((N_DEV - 1, ROWS, D), jnp.bfloat16),
            pltpu.SemaphoreType.DMA((N_DEV - 1,)),
            pltpu.SemaphoreType.DMA((N_DEV - 1,)),
            pltpu.SemaphoreType.DMA((2,)),
        ],
    )(x, Wq, Wo, K_ext, V_ext)


# device time: 45464 ns/iter; 4.6007x vs baseline; 1.5917x over previous
import jax
import jax.numpy as jnp
from jax import lax
from jax.experimental import pallas as pl
from jax.experimental.pallas import tpu as pltpu

N_DEV = 8
B, SQ, SKV, D = 2, 256, 512, 768
HQ_PER = 8
DH = 64
DM = HQ_PER * DH
SCALE = 0.125
ROWS = B * SQ
CH = ROWS // N_DEV

_MESH = pl.DeviceIdType.MESH
_ALLREDUCE = True


def kernel(x, Wq, Wo, K_ext, V_ext):
    my_sm = lax.axis_index("i")
    h0 = my_sm * HQ_PER
    Kmy = lax.dynamic_slice_in_dim(K_ext, h0, HQ_PER, axis=2)
    Vmy = lax.dynamic_slice_in_dim(V_ext, h0, HQ_PER, axis=2)
    Kmy = Kmy.astype(jnp.bfloat16).reshape(B, SKV, DM)
    Vmy = Vmy.astype(jnp.bfloat16).reshape(B, SKV, DM)

    def body(x_ref, wq_ref, wo_ref, k_ref, v_ref, out_ref,
             attn_ref, acc_ref, sbuf, cbuf, send_sems, recv_sems):
        my = lax.axis_index("i")
        r = my & 3
        myx = (r ^ (r >> 1)) & 1
        myy = r >> 1
        myz = my >> 2
        p_x = my ^ 1
        p_y = my ^ 3
        p_z = my ^ 4

        with jax.named_scope("qgemm"):
            x2 = x_ref[...].reshape(ROWS, D).astype(jnp.bfloat16)
            q_all = lax.dot_general(
                x2, wq_ref[...].astype(jnp.bfloat16),
                (((1,), (0,)), ((), ())),
                preferred_element_type=jnp.float32).astype(jnp.bfloat16)

        with jax.named_scope("attn"):
            for b in range(B):
                kb = k_ref[b]
                vb = v_ref[b]
                for h in range(HQ_PER):
                    q_bh = q_all[b * SQ:(b + 1) * SQ, h * DH:(h + 1) * DH]
                    k_bh = kb[:, h * DH:(h + 1) * DH]
                    v_bh = vb[:, h * DH:(h + 1) * DH]
                    s = lax.dot_general(
                        q_bh, k_bh, (((1,), (1,)), ((), ())),
                        preferred_element_type=jnp.float32) * SCALE
                    m = jnp.max(s, axis=1, keepdims=True)
                    p = jnp.exp(s - m)
                    l = jnp.sum(p, axis=1, keepdims=True)
                    o = lax.dot_general(
                        p.astype(jnp.bfloat16), v_bh, (((1,), (0,)), ((), ())),
                        preferred_element_type=jnp.float32)
                    attn_ref[pl.ds(b * SQ, SQ), pl.ds(h * DH, DH)] = (
                        o / l).astype(jnp.bfloat16)

        with jax.named_scope("ogemm"):
            acc_ref[...] = lax.dot_general(
                attn_ref[...], wo_ref[...].astype(jnp.bfloat16),
                (((1,), (0,)), ((), ())),
                preferred_element_type=jnp.float32)

        if not _ALLREDUCE:
            for b in range(B):
                out_ref[b] = acc_ref[b * SQ:(b + 1) * SQ, :]
            return

        pending = []

        def exchange(j, c_send, partner):
            sbuf[j] = acc_ref[pl.ds(c_send * CH, CH), :].astype(jnp.bfloat16)
            rdma = pltpu.make_async_remote_copy(
                src_ref=sbuf.at[j], dst_ref=cbuf.at[j],
                send_sem=send_sems.at[j], recv_sem=recv_sems.at[j],
                device_id=(partner,), device_id_type=_MESH)
            rdma.start()
            pending.append(rdma)

        def wait_recv(j):
            rdma = pltpu.make_async_remote_copy(
                src_ref=sbuf.at[0], dst_ref=cbuf.at[j],
                send_sem=send_sems.at[j], recv_sem=recv_sems.at[j],
                device_id=(my,), device_id_type=_MESH)
            rdma.wait_recv()

        with jax.named_scope("rs_x"):
            for k in range(4):
                exchange(k, (1 - myx) + 2 * k, p_x)
            for k in range(4):
                wait_recv(k)
                c = myx + 2 * k
                acc_ref[pl.ds(c * CH, CH), :] += cbuf[k].astype(jnp.float32)

        with jax.named_scope("rs_y"):
            for k in range(2):
                exchange(4 + k, myx + 2 * (1 - myy) + 4 * k, p_y)
            for k in range(2):
                wait_recv(4 + k)
                c = myx + 2 * myy + 4 * k
                acc_ref[pl.ds(c * CH, CH), :] += (
                    cbuf[4 + k].astype(jnp.float32))

        with jax.named_scope("rs_z"):
            exchange(6, myx + 2 * myy + 4 * (1 - myz), p_z)
            wait_recv(6)
            v_me = myx + 2 * myy + 4 * myz
            acc_ref[pl.ds(v_me * CH, CH), :] += cbuf[6].astype(jnp.float32)

        def forward(j, src):
            rdma = pltpu.make_async_remote_copy(
                src_ref=src, dst_ref=cbuf.at[j],
                send_sem=send_sems.at[j], recv_sem=recv_sems.at[j],
                device_id=(p_z if j == 7 else (p_y if j < 10 else p_x),),
                device_id_type=_MESH)
            rdma.start()
            pending.append(rdma)

        with jax.named_scope("ag_z"):
            sbuf[7] = acc_ref[pl.ds(v_me * CH, CH), :].astype(jnp.bfloat16)
            forward(7, sbuf.at[7])
            wait_recv(7)

        with jax.named_scope("ag_y"):
            forward(8, sbuf.at[7])
            forward(9, cbuf.at[7])
            wait_recv(8)
            wait_recv(9)

        with jax.named_scope("ag_x"):
            forward(10, sbuf.at[7])
            forward(11, cbuf.at[7])
            forward(12, cbuf.at[8])
            forward(13, cbuf.at[9])
            for j in range(10, 14):
                wait_recv(j)

        with jax.named_scope("store"):
            for slot, mask in ((7, 4), (8, 2), (9, 6), (10, 1), (11, 5),
                               (12, 3), (13, 7)):
                c = v_me ^ mask
                acc_ref[pl.ds(c * CH, CH), :] = cbuf[slot].astype(jnp.float32)
            for b in range(B):
                out_ref[b] = acc_ref[b * SQ:(b + 1) * SQ, :]
            for rdma in pending:
                rdma.wait_send()

    return pl.pallas_call(
        body,
        out_shape=jax.ShapeDtypeStruct((B, SQ, D), jnp.float32),
        in_specs=[
            pl.BlockSpec(memory_space=pltpu.MemorySpace.VMEM),
            pl.BlockSpec(memory_space=pltpu.MemorySpace.VMEM),
            pl.BlockSpec(memory_space=pltpu.MemorySpace.VMEM),
            pl.BlockSpec(memory_space=pltpu.MemorySpace.VMEM),
            pl.BlockSpec(memory_space=pltpu.MemorySpace.VMEM),
        ],
        out_specs=pl.BlockSpec(memory_space=pltpu.MemorySpace.VMEM),
        scratch_shapes=[
            pltpu.VMEM((ROWS, DM), jnp.bfloat16),
            pltpu.VMEM((ROWS, D), jnp.float32),
            pltpu.VMEM((8, CH, D), jnp.bfloat16),
            pltpu.VMEM((14, CH, D), jnp.bfloat16),
            pltpu.SemaphoreType.DMA((14,)),
            pltpu.SemaphoreType.DMA((14,)),
        ],
    )(x, Wq, Wo, Kmy, Vmy)


# device time: 34859 ns/iter; 6.0004x vs baseline; 1.3042x over previous
import jax
import jax.numpy as jnp
from jax import lax
from jax.experimental import pallas as pl
from jax.experimental.pallas import tpu as pltpu

N_DEV = 8
B, SQ, SKV, D = 2, 256, 512, 768
HQ_PER = 8
DH = 64
DM = HQ_PER * DH
SCALE = 0.125
ROWS = B * SQ
CH = ROWS // N_DEV

_MESH = pl.DeviceIdType.MESH
_ALLREDUCE = True


def kernel(x, Wq, Wo, K_ext, V_ext):
    my_sm = lax.axis_index("i")
    h0 = my_sm * HQ_PER
    Kmy = lax.dynamic_slice_in_dim(K_ext, h0, HQ_PER, axis=2)
    Vmy = lax.dynamic_slice_in_dim(V_ext, h0, HQ_PER, axis=2)
    Kmy = Kmy.astype(jnp.bfloat16).reshape(B, SKV, DM)
    Vmy = Vmy.astype(jnp.bfloat16).reshape(B, SKV, DM)

    def body(x_ref, wq_ref, wo_ref, k_ref, v_ref, out_ref,
             attn_ref, acc_ref, sbuf, cbuf, send_sems, recv_sems):
        my = lax.axis_index("i")
        r = my & 3
        myx = (r ^ (r >> 1)) & 1
        myy = r >> 1
        myz = my >> 2
        p_x = my ^ 1
        p_y = my ^ 3
        p_z = my ^ 4

        with jax.named_scope("qgemm"):
            x2 = x_ref[...].reshape(ROWS, D).astype(jnp.bfloat16)
            q_all = lax.dot_general(
                x2, wq_ref[...].astype(jnp.bfloat16),
                (((1,), (0,)), ((), ())),
                preferred_element_type=jnp.float32).astype(jnp.bfloat16)

        with jax.named_scope("attn"):
            for b in range(B):
                kb = k_ref[b]
                vb = v_ref[b]
                for h in range(HQ_PER):
                    q_bh = q_all[b * SQ:(b + 1) * SQ, h * DH:(h + 1) * DH]
                    k_bh = kb[:, h * DH:(h + 1) * DH]
                    v_bh = vb[:, h * DH:(h + 1) * DH]
                    s = lax.dot_general(
                        q_bh, k_bh, (((1,), (1,)), ((), ())),
                        preferred_element_type=jnp.float32) * SCALE
                    m = jnp.max(s, axis=1, keepdims=True)
                    p = jnp.exp(s - m)
                    l = jnp.sum(p, axis=1, keepdims=True)
                    o = lax.dot_general(
                        p.astype(jnp.bfloat16), v_bh, (((1,), (0,)), ((), ())),
                        preferred_element_type=jnp.float32)
                    attn_ref[pl.ds(b * SQ, SQ), pl.ds(h * DH, DH)] = (
                        o / l).astype(jnp.bfloat16)

        with jax.named_scope("ogemm"):
            acc_ref[...] = lax.dot_general(
                attn_ref[...], wo_ref[...].astype(jnp.bfloat16),
                (((1,), (0,)), ((), ())),
                preferred_element_type=jnp.float32)

        if not _ALLREDUCE:
            for b in range(B):
                out_ref[b] = acc_ref[b * SQ:(b + 1) * SQ, :]
            return

        pending = []

        def wait_recv(j):
            rdma = pltpu.make_async_remote_copy(
                src_ref=sbuf.at[0], dst_ref=cbuf.at[j],
                send_sem=send_sems.at[j], recv_sem=recv_sems.at[j],
                device_id=(my,), device_id_type=_MESH)
            rdma.wait_recv()

        with jax.named_scope("rs"):
            for d in range(1, N_DEV):
                p = lax.rem(my + d, N_DEV)
                sbuf[d - 1] = acc_ref[pl.ds(p * CH, CH), :].astype(
                    jnp.bfloat16)
                rdma = pltpu.make_async_remote_copy(
                    src_ref=sbuf.at[d - 1], dst_ref=cbuf.at[d - 1],
                    send_sem=send_sems.at[d - 1],
                    recv_sem=recv_sems.at[d - 1],
                    device_id=(p,), device_id_type=_MESH)
                rdma.start()
                pending.append(rdma)
            my_rows = pl.ds(my * CH, CH)
            for e in range(1, N_DEV):
                wait_recv(e - 1)
                acc_ref[my_rows, :] += cbuf[e - 1].astype(jnp.float32)

        with jax.named_scope("ag"):
            sbuf[7] = acc_ref[my_rows, :].astype(jnp.bfloat16)
            for d in range(1, N_DEV):
                p = lax.rem(my + d, N_DEV)
                rdma = pltpu.make_async_remote_copy(
                    src_ref=sbuf.at[7], dst_ref=cbuf.at[6 + d],
                    send_sem=send_sems.at[6 + d],
                    recv_sem=recv_sems.at[6 + d],
                    device_id=(p,), device_id_type=_MESH)
                rdma.start()
                pending.append(rdma)
            for e in range(1, N_DEV):
                wait_recv(6 + e)
                c = lax.rem(my + (N_DEV - e), N_DEV)
                acc_ref[pl.ds(c * CH, CH), :] = cbuf[6 + e].astype(
                    jnp.float32)

        with jax.named_scope("store"):
            for b in range(B):
                out_ref[b] = acc_ref[b * SQ:(b + 1) * SQ, :]
            for rdma in pending:
                rdma.wait_send()

    return pl.pallas_call(
        body,
        out_shape=jax.ShapeDtypeStruct((B, SQ, D), jnp.float32),
        in_specs=[
            pl.BlockSpec(memory_space=pltpu.MemorySpace.VMEM),
            pl.BlockSpec(memory_space=pltpu.MemorySpace.VMEM),
            pl.BlockSpec(memory_space=pltpu.MemorySpace.VMEM),
            pl.BlockSpec(memory_space=pltpu.MemorySpace.VMEM),
            pl.BlockSpec(memory_space=pltpu.MemorySpace.VMEM),
        ],
        out_specs=pl.BlockSpec(memory_space=pltpu.MemorySpace.VMEM),
        scratch_shapes=[
            pltpu.VMEM((ROWS, DM), jnp.bfloat16),
            pltpu.VMEM((ROWS, D), jnp.float32),
            pltpu.VMEM((8, CH, D), jnp.bfloat16),
            pltpu.VMEM((14, CH, D), jnp.bfloat16),
            pltpu.SemaphoreType.DMA((14,)),
            pltpu.SemaphoreType.DMA((14,)),
        ],
    )(x, Wq, Wo, Kmy, Vmy)
